# baseline (speedup 1.0000x reference)
import functools

import jax
import jax.numpy as jnp
from jax import lax
from jax.experimental import pallas as pl
from jax.experimental.pallas import tpu as pltpu

B, S, H = 64, 2048, 768
E, HID, L = 64, 512, 4

BB = 8
SB = 1024
G = 16


def _pool_body(h_ref, out_ref):
    j = pl.program_id(1)
    partial = jnp.sum(h_ref[...], axis=1)

    @pl.when(j == 0)
    def _():
        out_ref[...] = partial

    @pl.when(j > 0)
    def _():
        out_ref[...] = out_ref[...] + partial

    @pl.when(j == pl.num_programs(1) - 1)
    def _():
        pooled = out_ref[...] * (1.0 / S)
        mu = jnp.mean(pooled, axis=1, keepdims=True)
        var = jnp.mean((pooled - mu) ** 2, axis=1, keepdims=True)
        out_ref[...] = (pooled - mu) * jax.lax.rsqrt(var + 1e-5)


def _mlp_body(species_ref, xn_ref, g_ref, b_ref, w1_ref, b1_ref, w2_ref,
              b2_ref, out_ref):
    i = pl.program_id(0)

    acc = jnp.zeros((B, L), jnp.float32)
    xn = xn_ref[...]
    for g in range(G):
        e = i * G + g
        mask = species_ref[...] == e
        x = xn * g_ref[g, :, :] + b_ref[g, :, :]
        h = jnp.dot(x, w1_ref[g], preferred_element_type=jnp.float32)
        h = h + b1_ref[g, :, :]
        h = 0.5 * h * (1.0 + jax.lax.erf(h * 0.7071067811865476))
        logits = jax.lax.dot_general(
            h, w2_ref[g], (((1,), (1,)), ((), ())),
            preferred_element_type=jnp.float32)
        logits = logits + b2_ref[g, :, :]
        acc = acc + jnp.where(mask, logits, 0.0)

    @pl.when(i == 0)
    def _():
        out_ref[...] = acc

    @pl.when(i > 0)
    def _():
        out_ref[...] = out_ref[...] + acc


def kernel(hidden_states, species_idx, ln_g, ln_b, W1, b1, W2, b2):
    species_i32 = species_idx.astype(jnp.int32)

    xn = pl.pallas_call(
        _pool_body,
        grid=(B // BB, S // SB),
        in_specs=[pl.BlockSpec((BB, SB, H), lambda i, j: (i, j, 0))],
        out_specs=pl.BlockSpec((BB, H), lambda i, j: (i, 0)),
        out_shape=jax.ShapeDtypeStruct((B, H), jnp.float32),
    )(hidden_states)

    species2d = jnp.broadcast_to(species_i32.reshape(B, 1), (B, L))
    w2t = jnp.swapaxes(W2, 1, 2)

    logits = pl.pallas_call(
        _mlp_body,
        grid=(E // G,),
        in_specs=[
            pl.BlockSpec((B, L), lambda i: (0, 0)),
            pl.BlockSpec((B, H), lambda i: (0, 0)),
            pl.BlockSpec((G, 1, H), lambda i: (i, 0, 0)),
            pl.BlockSpec((G, 1, H), lambda i: (i, 0, 0)),
            pl.BlockSpec((G, H, HID), lambda i: (i, 0, 0)),
            pl.BlockSpec((G, 1, HID), lambda i: (i, 0, 0)),
            pl.BlockSpec((G, L, HID), lambda i: (i, 0, 0)),
            pl.BlockSpec((G, 1, L), lambda i: (i, 0, 0)),
        ],
        out_specs=pl.BlockSpec((B, L), lambda i: (0, 0)),
        out_shape=jax.ShapeDtypeStruct((B, L), jnp.float32),
    )(species2d, xn, ln_g.reshape(E, 1, H), ln_b.reshape(E, 1, H), W1,
      b1.reshape(E, 1, HID), w2t, b2.reshape(E, 1, L))
    return logits


# speedup vs baseline: 1.0065x; 1.0065x over previous
import jax
import jax.numpy as jnp
from jax import lax
from jax.experimental import pallas as pl
from jax.experimental.pallas import tpu as pltpu

B, S, H = 64, 2048, 768
E, HID, L = 64, 512, 4

BB = 8
SB = 512


def _pool_body(species_ref, h_ref, out_ref, meta_ref):
    i = pl.program_id(0)
    j = pl.program_id(1)
    partial = jnp.sum(h_ref[...], axis=1)

    @pl.when(j == 0)
    def _():
        out_ref[...] = partial

    @pl.when(j > 0)
    def _():
        out_ref[...] = out_ref[...] + partial

    @pl.when(j == pl.num_programs(1) - 1)
    def _():
        pooled = out_ref[...] * (1.0 / S)
        mu = jnp.mean(pooled, axis=1, keepdims=True)
        var = jnp.mean((pooled - mu) ** 2, axis=1, keepdims=True)
        out_ref[...] = (pooled - mu) * jax.lax.rsqrt(var + 1e-5)

    @pl.when((i == 0) & (j == 0))
    def _():
        row_e = lax.broadcasted_iota(jnp.int32, (E, B), 0)
        sp_row = species_ref[...]
        cmp = (jnp.broadcast_to(sp_row, (E, B)) == row_e)
        presence = jnp.max(cmp.astype(jnp.float32), axis=1,
                           keepdims=True)
        r0 = lax.broadcasted_iota(jnp.int32, (E, E), 0)
        c0 = lax.broadcasted_iota(jnp.int32, (E, E), 1)
        tril = (c0 < r0).astype(jnp.float32)
        pos = jnp.dot(tril, presence,
                      preferred_element_type=jnp.float32)
        num = jnp.sum(presence)
        col_i = c0.astype(jnp.float32)
        sel = (jnp.broadcast_to(pos, (E, E)) == col_i)
        selp = sel.astype(jnp.float32) * jnp.broadcast_to(
            presence, (E, E))
        e_col = r0.astype(jnp.float32)
        uids = jnp.sum(selp * e_col, axis=0, keepdims=True)
        padmax = jnp.max(presence * lax.broadcasted_iota(
            jnp.int32, (E, 1), 0).astype(jnp.float32))
        lane = lax.broadcasted_iota(
            jnp.int32, (1, E), 1).astype(jnp.float32)
        uids = jnp.where(lane < num, uids, padmax)
        meta = jnp.concatenate(
            [uids, jnp.full((1, E), num, jnp.float32)], axis=1)
        meta_ref[...] = meta.astype(jnp.int32)


def _mlp_body(meta_ref, species_ref, xn_ref, g_ref, b_ref, w1_ref, b1_ref,
              w2_ref, b2_ref, out_ref):
    i = pl.program_id(0)

    @pl.when(i == 0)
    def _():
        out_ref[...] = jnp.zeros_like(out_ref)

    @pl.when(i < meta_ref[E])
    def _():
        e = meta_ref[i]
        mask = species_ref[...] == e
        x = xn_ref[...] * g_ref[e, :, :] + b_ref[e, :, :]
        h = jnp.dot(x, w1_ref[0], preferred_element_type=jnp.float32)
        h = h + b1_ref[e, :, :]
        h = 0.5 * h * (1.0 + jax.lax.erf(h * 0.7071067811865476))
        logits = jax.lax.dot_general(
            h, w2_ref[e], (((1,), (1,)), ((), ())),
            preferred_element_type=jnp.float32)
        logits = logits + b2_ref[e, :, :]
        out_ref[...] = out_ref[...] + jnp.where(mask, logits, 0.0)


def kernel(hidden_states, species_idx, ln_g, ln_b, W1, b1, W2, b2):
    species_i32 = species_idx.astype(jnp.int32)
    species_row = species_i32.reshape(1, B)

    xn, meta2d = pl.pallas_call(
        _pool_body,
        grid=(B // BB, S // SB),
        in_specs=[
            pl.BlockSpec((1, B), lambda i, j: (0, 0)),
            pl.BlockSpec((BB, SB, H), lambda i, j: (i, j, 0)),
        ],
        out_specs=[
            pl.BlockSpec((BB, H), lambda i, j: (i, 0)),
            pl.BlockSpec((1, 2 * E), lambda i, j: (0, 0)),
        ],
        out_shape=[
            jax.ShapeDtypeStruct((B, H), jnp.float32),
            jax.ShapeDtypeStruct((1, 2 * E), jnp.int32),
        ],
    )(species_row, hidden_states)
    meta = meta2d.reshape(2 * E)

    species2d = jnp.broadcast_to(species_i32.reshape(B, 1), (B, L))
    w2t = jnp.swapaxes(W2, 1, 2)

    grid_spec = pltpu.PrefetchScalarGridSpec(
        num_scalar_prefetch=1,
        grid=(E,),
        in_specs=[
            pl.BlockSpec((B, L), lambda i, meta: (0, 0)),
            pl.BlockSpec((B, H), lambda i, meta: (0, 0)),
            pl.BlockSpec((E, 1, H), lambda i, meta: (0, 0, 0)),
            pl.BlockSpec((E, 1, H), lambda i, meta: (0, 0, 0)),
            pl.BlockSpec((1, H, HID), lambda i, meta: (meta[i], 0, 0)),
            pl.BlockSpec((E, 1, HID), lambda i, meta: (0, 0, 0)),
            pl.BlockSpec((E, L, HID), lambda i, meta: (0, 0, 0)),
            pl.BlockSpec((E, 1, L), lambda i, meta: (0, 0, 0)),
        ],
        out_specs=pl.BlockSpec((B, L), lambda i, meta: (0, 0)),
    )

    logits = pl.pallas_call(
        _mlp_body,
        grid_spec=grid_spec,
        out_shape=jax.ShapeDtypeStruct((B, L), jnp.float32),
    )(meta, species2d, xn, ln_g.reshape(E, 1, H), ln_b.reshape(E, 1, H), W1,
      b1.reshape(E, 1, HID), w2t, b2.reshape(E, 1, L))
    return logits


# speedup vs baseline: 1.0918x; 1.0848x over previous
import jax
import jax.numpy as jnp
from jax import lax
from jax.experimental import pallas as pl
from jax.experimental.pallas import tpu as pltpu

B, S, H = 64, 2048, 768
E, HID, L = 64, 512, 4

BB = 8
SB = 512
K = 4


def _pool_body(species_ref, h_ref, out_ref, meta_ref):
    i = pl.program_id(0)
    j = pl.program_id(1)
    partial = jnp.sum(h_ref[...], axis=1)

    @pl.when(j == 0)
    def _():
        out_ref[...] = partial

    @pl.when(j > 0)
    def _():
        out_ref[...] = out_ref[...] + partial

    @pl.when(j == pl.num_programs(1) - 1)
    def _():
        pooled = out_ref[...] * (1.0 / S)
        mu = jnp.mean(pooled, axis=1, keepdims=True)
        var = jnp.mean((pooled - mu) ** 2, axis=1, keepdims=True)
        out_ref[...] = (pooled - mu) * jax.lax.rsqrt(var + 1e-5)

    @pl.when((i == 0) & (j == 0))
    def _():
        row_e = lax.broadcasted_iota(jnp.int32, (E, B), 0)
        sp_row = species_ref[...]
        cmp = (jnp.broadcast_to(sp_row, (E, B)) == row_e)
        presence = jnp.max(cmp.astype(jnp.float32), axis=1,
                           keepdims=True)
        r0 = lax.broadcasted_iota(jnp.int32, (E, E), 0)
        c0 = lax.broadcasted_iota(jnp.int32, (E, E), 1)
        tril = (c0 < r0).astype(jnp.float32)
        pos = jnp.dot(tril, presence,
                      preferred_element_type=jnp.float32)
        num = jnp.sum(presence)
        col_i = c0.astype(jnp.float32)
        sel = (jnp.broadcast_to(pos, (E, E)) == col_i)
        selp = sel.astype(jnp.float32) * jnp.broadcast_to(
            presence, (E, E))
        e_col = r0.astype(jnp.float32)
        uids = jnp.sum(selp * e_col, axis=0, keepdims=True)
        padmax = jnp.max(presence * lax.broadcasted_iota(
            jnp.int32, (E, 1), 0).astype(jnp.float32))
        lane = lax.broadcasted_iota(
            jnp.int32, (1, E), 1).astype(jnp.float32)
        uids = jnp.where(lane < num, uids, padmax)
        meta = jnp.concatenate(
            [uids, jnp.full((1, E), num, jnp.float32)], axis=1)
        meta_ref[...] = meta.astype(jnp.int32)


def _mlp_body(meta_ref, species_ref, xn_ref, g_ref, b_ref, w1a_ref, w1b_ref,
              w1c_ref, w1d_ref, b1_ref, w2_ref, b2_ref, out_ref):
    i = pl.program_id(0)

    @pl.when(i == 0)
    def _():
        out_ref[...] = jnp.zeros_like(out_ref)

    xn = xn_ref[...]
    for k, w1_ref in enumerate((w1a_ref, w1b_ref, w1c_ref, w1d_ref)):
        @pl.when(i * K + k < meta_ref[E])
        def _(k=k, w1_ref=w1_ref):
            e = meta_ref[i * K + k]
            mask = species_ref[...] == e
            x = xn * g_ref[e, :, :] + b_ref[e, :, :]
            h = jnp.dot(x, w1_ref[0], preferred_element_type=jnp.float32)
            h = h + b1_ref[e, :, :]
            h = 0.5 * h * (1.0 + jax.lax.erf(h * 0.7071067811865476))
            logits = jax.lax.dot_general(
                h, w2_ref[e], (((1,), (1,)), ((), ())),
                preferred_element_type=jnp.float32)
            logits = logits + b2_ref[e, :, :]
            out_ref[...] = out_ref[...] + jnp.where(mask, logits, 0.0)


def kernel(hidden_states, species_idx, ln_g, ln_b, W1, b1, W2, b2):
    species_i32 = species_idx.astype(jnp.int32)
    species_row = species_i32.reshape(1, B)

    xn, meta2d = pl.pallas_call(
        _pool_body,
        grid=(B // BB, S // SB),
        in_specs=[
            pl.BlockSpec((1, B), lambda i, j: (0, 0)),
            pl.BlockSpec((BB, SB, H), lambda i, j: (i, j, 0)),
        ],
        out_specs=[
            pl.BlockSpec((BB, H), lambda i, j: (i, 0)),
            pl.BlockSpec((1, 2 * E), lambda i, j: (0, 0)),
        ],
        out_shape=[
            jax.ShapeDtypeStruct((B, H), jnp.float32),
            jax.ShapeDtypeStruct((1, 2 * E), jnp.int32),
        ],
    )(species_row, hidden_states)
    meta = meta2d.reshape(2 * E)

    species2d = jnp.broadcast_to(species_i32.reshape(B, 1), (B, L))
    w2t = jnp.swapaxes(W2, 1, 2)

    def w1_lane(k):
        return pl.BlockSpec((1, H, HID), lambda i, meta: (meta[i * K + k],
                                                          0, 0))

    grid_spec = pltpu.PrefetchScalarGridSpec(
        num_scalar_prefetch=1,
        grid=(E // K,),
        in_specs=[
            pl.BlockSpec((B, L), lambda i, meta: (0, 0)),
            pl.BlockSpec((B, H), lambda i, meta: (0, 0)),
            pl.BlockSpec((E, 1, H), lambda i, meta: (0, 0, 0)),
            pl.BlockSpec((E, 1, H), lambda i, meta: (0, 0, 0)),
            w1_lane(0),
            w1_lane(1),
            w1_lane(2),
            w1_lane(3),
            pl.BlockSpec((E, 1, HID), lambda i, meta: (0, 0, 0)),
            pl.BlockSpec((E, L, HID), lambda i, meta: (0, 0, 0)),
            pl.BlockSpec((E, 1, L), lambda i, meta: (0, 0, 0)),
        ],
        out_specs=pl.BlockSpec((B, L), lambda i, meta: (0, 0)),
    )

    logits = pl.pallas_call(
        _mlp_body,
        grid_spec=grid_spec,
        out_shape=jax.ShapeDtypeStruct((B, L), jnp.float32),
    )(meta, species2d, xn, ln_g.reshape(E, 1, H), ln_b.reshape(E, 1, H),
      W1, W1, W1, W1,
      b1.reshape(E, 1, HID), w2t, b2.reshape(E, 1, L))
    return logits
